# baseline (speedup 1.0000x reference)
import jax
import jax.numpy as jnp
from jax.experimental import pallas as pl


def _mlp(ps, x):
    h = x @ ps[0]["W"] + ps[0]["b"]
    h = jax.nn.relu(h)
    return h @ ps[1]["W"] + ps[1]["b"]


def kernel(x, edge_index, edge_attr, u, batch, params):
    row = edge_index[0]
    col = edge_index[1]
    h, ea, uu = x, edge_attr, u
    Nn = x.shape[0]
    Gg = u.shape[0]
    for i, lp in enumerate(params["layers"]):
        e_in = jnp.concatenate([h[row], h[col], ea, uu[batch[row]]], axis=1)
        e_out = _mlp(lp["edge"], e_in)
        if i > 0:
            e_out = e_out + ea
        ea = e_out
        s_add = jax.ops.segment_sum(ea, col, num_segments=Nn)
        s_max = jax.ops.segment_max(ea, col, num_segments=Nn)
        s_max = jnp.where(jnp.isfinite(s_max), s_max, 0.0)
        cnt = jax.ops.segment_sum(jnp.ones((ea.shape[0],), dtype=ea.dtype), col, num_segments=Nn)
        s_mean = s_add / jnp.maximum(cnt, 1.0)[:, None]
        n_in = jnp.concatenate([h, s_add, s_max, s_mean, uu[batch]], axis=1)
        n_out = _mlp(lp["node"], n_in)
        if i > 0:
            n_out = n_out + h
        h = n_out
        g_sum = jax.ops.segment_sum(h, batch, num_segments=Gg)
        g_cnt = jax.ops.segment_sum(jnp.ones((Nn,), dtype=h.dtype), batch, num_segments=Gg)
        g_mean = g_sum / jnp.maximum(g_cnt, 1.0)[:, None]
        g_in = jnp.concatenate([uu, g_mean], axis=1)
        g_out = _mlp(lp["global"], g_in)
        if i > 0:
            g_out = g_out + uu
        uu = g_out
    addpool = jax.ops.segment_sum(h, batch, num_segments=Gg)
    g_cnt = jax.ops.segment_sum(jnp.ones((Nn,), dtype=h.dtype), batch, num_segments=Gg)
    meanpool = addpool / jnp.maximum(g_cnt, 1.0)[:, None]
    maxpool = jax.ops.segment_max(h, batch, num_segments=Gg)
    maxpool = jnp.where(jnp.isfinite(maxpool), maxpool, 0.0)
    out = jnp.concatenate([addpool, meanpool, maxpool, uu], axis=1)
    op = params["out"]
    o = jax.nn.relu(out @ op[0]["W"] + op[0]["b"])
    o = jax.nn.relu(o @ op[1]["W"] + op[1]["b"])
    o = o @ op[2]["W"] + op[2]["b"]
    return jax.nn.sigmoid(o)


# speedup vs baseline: 1.3813x; 1.3813x over previous
import functools

import jax
import jax.numpy as jnp
from jax import lax
from jax.experimental import pallas as pl
from jax.experimental.pallas import tpu as pltpu
from jax.experimental.pallas import tpu_sc as plsc

_N = 10000
_E = 320000
_G = 8
_DOUT = 6
_NC = 2
_NS = 16
_NW = _NC * _NS
_EPT = _E // _NW
_NPT = ((-(-_N // _NW)) + 7) // 8 * 8
_NPAD = _NW * _NPT
_BLK = 128
_CAP = _E + _BLK
_CS = 2000
_IDCAP = 2304
_GC = 80
_NB = 1000

_mesh = plsc.VectorSubcoreMesh(
    core_axis_name="c", subcore_axis_name="s", num_cores=_NC, num_subcores=_NS)

_HI = lax.Precision.HIGHEST


def _dot(a, b):
    return jnp.dot(a, b, precision=_HI, preferred_element_type=jnp.float32)


def _wid():
    return lax.axis_index("s") * _NC + lax.axis_index("c")


def _gather_body(t_hbm, row_hbm, col_hbm, pre_hbm,
                 ib1, ib2, g1, g2, sem1, sem2):
    base = _wid() * _EPT

    def _chunk(ci, _):
        off = pl.multiple_of(base + ci * _GC, 16)
        pltpu.sync_copy(row_hbm.at[pl.ds(off, _GC)], ib1)
        pltpu.sync_copy(col_hbm.at[pl.ds(off, _GC)], ib2)
        c1 = pltpu.async_copy(t_hbm.at[ib1], g1, sem1)
        c2 = pltpu.async_copy(t_hbm.at[ib2], g2, sem2)
        c1.wait()
        c2.wait()

        def _add(r, _):
            for q in range(4):
                a = g1[r, pl.ds(q * 16, 16)]
                b = g2[r, pl.ds(64 + q * 16, 16)]
                g1[r, pl.ds(q * 16, 16)] = a + b
            return 0

        lax.fori_loop(0, _GC, _add, 0)
        pltpu.sync_copy(g1, pre_hbm.at[pl.ds(off, _GC)])
        return 0

    lax.fori_loop(0, _EPT // _GC, _chunk, 0)


def _run_gather(t, row, col):
    f = pl.kernel(
        _gather_body,
        out_type=jax.ShapeDtypeStruct((_E, 128), jnp.float32),
        mesh=_mesh,
        scratch_types=[pltpu.VMEM((_GC,), jnp.int32),
                       pltpu.VMEM((_GC,), jnp.int32),
                       pltpu.VMEM((_GC, 128), jnp.float32),
                       pltpu.VMEM((_GC, 128), jnp.float32),
                       pltpu.SemaphoreType.DMA,
                       pltpu.SemaphoreType.DMA],
    )
    return f(t, row, col)


_EB = 512


def _scatter_body(col_ref, eout_ref, sadd_ref, smax_ref, cnt_ref):
    i = pl.program_id(0)

    @pl.when(i == 0)
    def _():
        sadd_ref[...] = jnp.zeros_like(sadd_ref)
        smax_ref[...] = jnp.full_like(smax_ref, -jnp.inf)
        cnt_ref[...] = jnp.zeros_like(cnt_ref)

    def _edge(e, _):
        c = col_ref[e]
        row = eout_ref[pl.ds(e, 1), pl.ds(0, 64)]
        sadd_ref[pl.ds(c, 1), :] += row
        smax_ref[pl.ds(c, 1), :] = jnp.maximum(smax_ref[pl.ds(c, 1), :], row)
        cnt_ref[pl.ds(c, 1), :] += 1.0
        return 0

    lax.fori_loop(0, _EB, _edge, 0)


def _run_scatter(eout, col):
    full = lambda s: pl.BlockSpec(s, lambda i: (0, 0))
    return pl.pallas_call(
        _scatter_body,
        grid=(_E // _EB,),
        in_specs=[pl.BlockSpec((_EB,), lambda i: (i,),
                               memory_space=pltpu.SMEM),
                  pl.BlockSpec((_EB, 128), lambda i: (i, 0))],
        out_specs=[full((_N, 64)), full((_N, 64)), full((_N, 8))],
        out_shape=[jax.ShapeDtypeStruct((_N, 64), jnp.float32),
                   jax.ShapeDtypeStruct((_N, 64), jnp.float32),
                   jax.ShapeDtypeStruct((_N, 8), jnp.float32)],
    )(col, eout)


def _proj_body(h_ref, b2d_ref, uu_ref, ws_ref, wd_ref, wu_ref, b1_ref,
               t_ref):
    hb = h_ref[...]
    oh = (b2d_ref[...] == lax.broadcasted_iota(jnp.int32, (1, _G), 1)
          ).astype(jnp.float32)
    ug = _dot(uu_ref[...], wu_ref[...])
    p = _dot(hb, ws_ref[...]) + _dot(oh, ug) + b1_ref[...][0:1]
    q = _dot(hb, wd_ref[...])
    t_ref[...] = jnp.concatenate([p, q], axis=1)


def _run_proj(h, batch2d, uu, ws, wd, wu, b1):
    dh = h.shape[1]
    du = uu.shape[1]
    full = lambda s: pl.BlockSpec(s, lambda i: (0, 0))
    return pl.pallas_call(
        _proj_body,
        grid=(_N // _NB,),
        in_specs=[pl.BlockSpec((_NB, dh), lambda i: (i, 0)),
                  pl.BlockSpec((_NB, 1), lambda i: (i, 0)),
                  full((_G, du)), full((dh, 64)), full((dh, 64)),
                  full((du, 64)), full((8, 64))],
        out_specs=pl.BlockSpec((_NB, 128), lambda i: (i, 0)),
        out_shape=jax.ShapeDtypeStruct((_N, 128), jnp.float32),
    )(h, batch2d, uu, ws, wd, wu, b1)


def _edge_body(residual, de, pre_ref, ea_ref, we_ref, w2_ref, b2_ref,
               out_ref):
    ea = ea_ref[...][:, :de]
    hid = jnp.maximum(pre_ref[...][:, :64] + _dot(ea, we_ref[...]), 0.0)
    o = _dot(hid, w2_ref[...]) + b2_ref[...][0:1]
    if residual:
        o = o + ea
    out_ref[...] = jnp.concatenate([o, jnp.zeros_like(o)], axis=1)


def _run_edge(pre, ea, we, w2, b2, residual):
    de = we.shape[0]
    eaw = ea.shape[1]
    eb = 2000
    full = lambda s: pl.BlockSpec(s, lambda i: (0, 0))
    return pl.pallas_call(
        functools.partial(_edge_body, residual, de),
        grid=(_E // eb,),
        in_specs=[pl.BlockSpec((eb, 128), lambda i: (i, 0)),
                  pl.BlockSpec((eb, eaw), lambda i: (i, 0)),
                  full((de, 64)), full((64, 64)), full((8, 64))],
        out_specs=pl.BlockSpec((eb, 128), lambda i: (i, 0)),
        out_shape=jax.ShapeDtypeStruct((_E, 128), jnp.float32),
    )(pre, ea, we, w2, b2)


def _node_body(mode, h_ref, sadd_ref, smax_ref, cnt_ref, b2d_ref, uu_ref,
               wh_ref, wa_ref, wm_ref, wme_ref, wun_ref, b1_ref, w2_ref,
               b2_ref, wgu_ref, wgm_ref, bg1_ref, wg2_ref, bg2_ref,
               o0_ref, ob0_ref, o1_ref, ob1_ref, o2_ref, ob2_ref,
               nout_ref, aadd_ref, amax_ref, acnt_ref, fin_ref):
    i = pl.program_id(0)
    nsteps = pl.num_programs(0)
    h = h_ref[...]
    sadd = sadd_ref[...]
    smaxr = smax_ref[...]
    smax = jnp.where(jnp.isfinite(smaxr), smaxr, 0.0)
    cnt = cnt_ref[...][:, 0:1]
    smean = sadd / jnp.maximum(cnt, 1.0)
    bt2 = b2d_ref[...]
    oh = (bt2 == lax.broadcasted_iota(jnp.int32, (1, _G), 1)
          ).astype(jnp.float32)
    uu = uu_ref[...]
    ugn = _dot(uu, wun_ref[...])
    hid = jnp.maximum(
        _dot(h, wh_ref[...]) + _dot(sadd, wa_ref[...]) +
        _dot(smax, wm_ref[...]) + _dot(smean, wme_ref[...]) +
        _dot(oh, ugn) + b1_ref[...][0:1], 0.0)
    no = _dot(hid, w2_ref[...]) + b2_ref[...][0:1]
    if mode == 2:
        no = no + h
    nout_ref[...] = no

    badd = lax.dot_general(oh, no, (((0,), (0,)), ((), ())),
                           precision=_HI, preferred_element_type=jnp.float32)
    parts = []
    for g in range(_G):
        mg = jnp.max(jnp.where(bt2 == g, no, -jnp.inf),
                     axis=0, keepdims=True)
        parts.append(mg)
    bmax = jnp.concatenate(parts, axis=0)
    bcnt = jnp.broadcast_to(jnp.sum(oh, axis=0)[:, None], (_G, 64))

    @pl.when(i == 0)
    def _():
        aadd_ref[...] = badd
        amax_ref[...] = bmax
        acnt_ref[...] = bcnt

    @pl.when(i > 0)
    def _():
        aadd_ref[...] += badd
        amax_ref[...] = jnp.maximum(amax_ref[...], bmax)
        acnt_ref[...] += bcnt

    @pl.when(i == nsteps - 1)
    def _():
        gadd = aadd_ref[...]
        gcnt = acnt_ref[...][:, 0:1]
        gmean = gadd / jnp.maximum(gcnt, 1.0)
        ghid = jnp.maximum(
            _dot(uu, wgu_ref[...]) + _dot(gmean, wgm_ref[...]) +
            bg1_ref[...][0:1], 0.0)
        go = _dot(ghid, wg2_ref[...]) + bg2_ref[...][0:1]
        if mode == 1:
            fin_ref[...] = go
        else:
            uu2 = go + uu
            gmaxr = amax_ref[...]
            gmax = jnp.where(jnp.isfinite(gmaxr), gmaxr, 0.0)
            cat = jnp.concatenate([gadd, gmean, gmax, uu2], axis=1)
            o = jnp.maximum(_dot(cat, o0_ref[...]) + ob0_ref[...][0:1], 0.0)
            o = jnp.maximum(_dot(o, o1_ref[...]) + ob1_ref[...][0:1], 0.0)
            o = _dot(o, o2_ref[...]) + ob2_ref[...][0:1]
            fin_ref[...] = jax.nn.sigmoid(o)


def _run_node(mode, h, sadd, smax, cnt, batch2d, uu, wh, wa, wm, wme, wun,
              b1, w2, b2, wgu, wgm, bg1, wg2, bg2, o0, ob0, o1, ob1, o2, ob2):
    dh = h.shape[1]
    du = uu.shape[1]
    full = lambda s: pl.BlockSpec(s, lambda i: (0, 0))
    fin_shape = (_G, 64) if mode == 1 else (_G, _DOUT)
    return pl.pallas_call(
        functools.partial(_node_body, mode),
        grid=(_N // _NB,),
        in_specs=[pl.BlockSpec((_NB, dh), lambda i: (i, 0)),
                  pl.BlockSpec((_NB, 64), lambda i: (i, 0)),
                  pl.BlockSpec((_NB, 64), lambda i: (i, 0)),
                  pl.BlockSpec((_NB, 8), lambda i: (i, 0)),
                  pl.BlockSpec((_NB, 1), lambda i: (i, 0)),
                  full((_G, du)),
                  full((dh, 64)), full((64, 64)), full((64, 64)),
                  full((64, 64)), full((du, 64)), full((8, 64)),
                  full((64, 64)), full((8, 64)),
                  full((du, 64)), full((64, 64)), full((8, 64)),
                  full((64, 64)), full((8, 64)),
                  full((256, 64)), full((8, 64)), full((64, 64)),
                  full((8, 64)), full((64, _DOUT)), full((8, _DOUT))],
        out_specs=[pl.BlockSpec((_NB, 64), lambda i: (i, 0)),
                   full((_G, 64)), full((_G, 64)), full((_G, 64)),
                   full(fin_shape)],
        out_shape=[jax.ShapeDtypeStruct((_N, 64), jnp.float32),
                   jax.ShapeDtypeStruct((_G, 64), jnp.float32),
                   jax.ShapeDtypeStruct((_G, 64), jnp.float32),
                   jax.ShapeDtypeStruct((_G, 64), jnp.float32),
                   jax.ShapeDtypeStruct(fin_shape, jnp.float32)],
    )(h, sadd, smax, cnt, batch2d, uu, wh, wa, wm, wme, wun, b1, w2, b2,
      wgu, wgm, bg1, wg2, bg2, o0, ob0, o1, ob1, o2, ob2)


def _b8(b):
    return jnp.broadcast_to(b.reshape(1, -1), (8, b.shape[0]))


def kernel(x, edge_index, edge_attr, u, batch, params):
    row = edge_index[0]
    col = edge_index[1]
    batch2d = batch.reshape(_N, 1)


    h, ea, uu = x, edge_attr, u
    out = None
    for li, lp in enumerate(params["layers"]):
        dh = h.shape[1]
        du = uu.shape[1]
        de = 16 if li == 0 else 64
        ew1, eb1 = lp["edge"][0]["W"], lp["edge"][0]["b"]
        ew2, eb2 = lp["edge"][1]["W"], lp["edge"][1]["b"]
        ws, wd = ew1[:dh], ew1[dh:2 * dh]
        we, wu = ew1[2 * dh:2 * dh + de], ew1[2 * dh + de:]

        t = _run_proj(h, batch2d, uu, ws, wd, wu, _b8(eb1))
        pre = _run_gather(t, row, col)
        eout = _run_edge(pre, ea, we, ew2, _b8(eb2), residual=(li > 0))
        sadd, smax, cnt = _run_scatter(eout, col)

        nw1, nb1 = lp["node"][0]["W"], lp["node"][0]["b"]
        nw2, nb2 = lp["node"][1]["W"], lp["node"][1]["b"]
        wh, wa = nw1[:dh], nw1[dh:dh + 64]
        wm, wme = nw1[dh + 64:dh + 128], nw1[dh + 128:dh + 192]
        wun = nw1[dh + 192:]
        gw1, gb1 = lp["global"][0]["W"], lp["global"][0]["b"]
        gw2, gb2 = lp["global"][1]["W"], lp["global"][1]["b"]
        wgu, wgm = gw1[:du], gw1[du:]

        op = params["out"]
        nout, _, _, _, fin = _run_node(
            1 if li == 0 else 2, h, sadd, smax, cnt, batch2d, uu,
            wh, wa, wm, wme, wun, _b8(nb1), nw2, _b8(nb2),
            wgu, wgm, _b8(gb1), gw2, _b8(gb2),
            op[0]["W"], _b8(op[0]["b"]), op[1]["W"], _b8(op[1]["b"]),
            op[2]["W"], _b8(op[2]["b"]))

        h, ea = nout, eout
        if li == 0:
            uu = fin
        else:
            out = fin
    return out
